# Initial kernel scaffold; baseline (speedup 1.0000x reference)
import jax
import jax.numpy as jnp
from jax.experimental import pallas as pl


def kernel(x, edge_index, Wl1, bl1, Wr1, Wl2, bl2, Wr2):
    raise NotImplementedError("write your pallas kernel here")


# speedup vs baseline: 5.1375x; 5.1375x over previous
import dataclasses
import functools

import jax
import jax.numpy as jnp
from jax import lax
from jax.experimental import pallas as pl
from jax.experimental.pallas import tpu as pltpu
from jax.experimental.pallas import tpu_sc as plsc

N = 10000
E = 320000
D = 128

NUM_CORES = 2
NUM_SUBCORES = 16
NUM_TILES = NUM_CORES * NUM_SUBCORES
CHUNK = 80
CHUNKS_PER_TILE = E // (NUM_TILES * CHUNK)
N_PAD = 10240
ROWS_PER_TILE = N_PAD // NUM_SUBCORES
ZROWS = 128


def _sc_agg_body(x_hbm, src_hbm, dst_hbm, agg_out,
                 idx_src, idx_dst, rows, zbuf, agg_sh):
    cid = lax.axis_index("c")
    sid = lax.axis_index("s")
    wid = cid * NUM_SUBCORES + sid

    zero16 = jnp.zeros((16,), jnp.float32)

    @pl.loop(0, ZROWS)
    def _(r):
        @pl.loop(0, D // 16)
        def _(c):
            zbuf[r, pl.ds(c * 16, 16)] = zero16

    @pl.loop(0, ROWS_PER_TILE // ZROWS)
    def _(k):
        pltpu.sync_copy(zbuf, agg_sh.at[pl.ds(sid * ROWS_PER_TILE + k * ZROWS, ZROWS)])

    plsc.subcore_barrier()

    @pl.loop(0, CHUNKS_PER_TILE)
    def _(j):
        base = (wid * CHUNKS_PER_TILE + j) * CHUNK
        pltpu.sync_copy(src_hbm.at[pl.ds(base, CHUNK)], idx_src)
        pltpu.sync_copy(dst_hbm.at[pl.ds(base, CHUNK)], idx_dst)
        pltpu.sync_copy(x_hbm.at[idx_src], rows)
        pltpu.sync_copy(rows, agg_sh.at[idx_dst], add=True)

    plsc.subcore_barrier()

    pltpu.sync_copy(agg_sh.at[pl.ds(sid * ROWS_PER_TILE, ROWS_PER_TILE)],
                    agg_out.at[cid, pl.ds(sid * ROWS_PER_TILE, ROWS_PER_TILE)])


def _sc_count_body(dst_hbm, cnt_out, idx_dst, cnt_local):
    cid = lax.axis_index("c")
    sid = lax.axis_index("s")
    wid = cid * NUM_SUBCORES + sid

    zero16 = jnp.zeros((16,), jnp.float32)
    zeros16i = jnp.zeros((16,), jnp.int32)
    ones16 = jnp.ones((16,), jnp.float32)

    @pl.loop(0, N // 16)
    def _(r):
        cnt_local[0, pl.ds(r * 16, 16)] = zero16

    @pl.loop(0, CHUNKS_PER_TILE)
    def _(j):
        base = (wid * CHUNKS_PER_TILE + j) * CHUNK
        pltpu.sync_copy(dst_hbm.at[pl.ds(base, CHUNK)], idx_dst)

        @pl.loop(0, CHUNK // 16)
        def _(t):
            idx16 = idx_dst[pl.ds(t * 16, 16)]
            plsc.addupdate_scatter(cnt_local, [zeros16i, idx16], ones16)

    pltpu.sync_copy(cnt_local, cnt_out.at[wid])


_SC_MESH = plsc.VectorSubcoreMesh(core_axis_name="c", subcore_axis_name="s")

_sc_agg = pl.kernel(
    _sc_agg_body,
    out_type=jax.ShapeDtypeStruct((NUM_CORES, N_PAD, D), jnp.float32),
    mesh=_SC_MESH,
    scratch_types=[
        pltpu.VMEM((CHUNK,), jnp.int32),
        pltpu.VMEM((CHUNK,), jnp.int32),
        pltpu.VMEM((CHUNK, D), jnp.float32),
        pltpu.VMEM((ZROWS, D), jnp.float32),
        pltpu.VMEM_SHARED((N_PAD, D), jnp.float32),
    ],
)

_SC_CP = pltpu.CompilerParams()
if "needs_layout_passes" in pltpu.CompilerParams.__dataclass_fields__:
    _SC_CP = dataclasses.replace(_SC_CP, needs_layout_passes=False)

_sc_count = pl.kernel(
    _sc_count_body,
    out_type=jax.ShapeDtypeStruct((NUM_TILES, 1, N), jnp.float32),
    mesh=_SC_MESH,
    scratch_types=[
        pltpu.VMEM((CHUNK,), jnp.int32),
        pltpu.VMEM((1, N), jnp.float32),
    ],
    compiler_params=_SC_CP,
)

_RB = 400


def _finish_body(relu, agg_ref, cnt_ref, x_ref, wl_ref, bl_ref, wr_ref, o_ref):
    agg = agg_ref[0] + agg_ref[1]
    cnt = jnp.sum(cnt_ref[...], axis=1, keepdims=True)
    mean = agg * (1.0 / jnp.maximum(cnt, 1.0))
    z = lax.dot_general(mean, wl_ref[...], (((1,), (1,)), ((), ())),
                        preferred_element_type=jnp.float32)
    z = z + bl_ref[...]
    z = z + lax.dot_general(x_ref[...], wr_ref[...], (((1,), (1,)), ((), ())),
                            preferred_element_type=jnp.float32)
    o_ref[...] = jnp.maximum(z, 0.0) if relu else z


def _make_finish(relu):
    return pl.pallas_call(
        functools.partial(_finish_body, relu),
        grid=(N // _RB,),
        in_specs=[
            pl.BlockSpec((NUM_CORES, _RB, D), lambda i: (0, i, 0)),
            pl.BlockSpec((_RB, NUM_TILES), lambda i: (i, 0)),
            pl.BlockSpec((_RB, D), lambda i: (i, 0)),
            pl.BlockSpec((D, D), lambda i: (0, 0)),
            pl.BlockSpec((1, D), lambda i: (0, 0)),
            pl.BlockSpec((D, D), lambda i: (0, 0)),
        ],
        out_specs=pl.BlockSpec((_RB, D), lambda i: (i, 0)),
        out_shape=jax.ShapeDtypeStruct((N, D), jnp.float32),
    )


_finish_relu = _make_finish(True)
_finish_plain = _make_finish(False)


def kernel(x, edge_index, Wl1, bl1, Wr1, Wl2, bl2, Wr2):
    src = edge_index[0]
    dst = edge_index[1]
    cntT = _sc_count(dst).reshape(NUM_TILES, N).T
    agg1 = _sc_agg(x, src, dst)
    h = _finish_relu(agg1, cntT, x, Wl1, bl1.reshape(1, D), Wr1)
    agg2 = _sc_agg(h, src, dst)
    return _finish_plain(agg2, cntT, h, Wl2, bl2.reshape(1, D), Wr2)
